# baseline (speedup 1.0000x reference)
import functools
import jax
import jax.numpy as jnp
from jax import lax
from jax.experimental import pallas as pl
from jax.experimental.pallas import tpu as pltpu
from jax.experimental.pallas import tpu_sc as plsc

N = 10000
E = 320000
D = 128

NC = 2
NS = 16
NW = NC * NS

CH = 128
NCHUNK = 80
EP = NW * NCHUNK * CH
NBUF = 2
N_PAD = 10112
ROWS_PER_TILE = N_PAD // NS
DUMP_SIZES = (128, 128, 128, 128, 120)
NSLOP = N_PAD - N

_mesh = plsc.VectorSubcoreMesh(core_axis_name="c", subcore_axis_name="s",
                               num_cores=NC, num_subcores=NS)


@functools.partial(
    pl.kernel,
    out_type=jax.ShapeDtypeStruct((NC * N_PAD,), jnp.float32),
    mesh=_mesh,
    scratch_types=[
        pltpu.VMEM_SHARED((N_PAD,), jnp.float32),
        pltpu.VMEM((NCHUNK, CH), jnp.int32),
        pltpu.VMEM((CH,), jnp.float32),
        pltpu.VMEM((ROWS_PER_TILE,), jnp.float32),
        pltpu.SemaphoreType.DMA,
    ],
)
def _sc_degree(dstc_hbm, ones_hbm, zvec_hbm, deg_out, acc_sh, slab_v, ones_v,
               zbuf_v, sem):
    c = lax.axis_index("c")
    s = lax.axis_index("s")
    wid = c * NS + s
    pltpu.sync_copy(zvec_hbm, zbuf_v)
    pltpu.sync_copy(zbuf_v, acc_sh.at[pl.ds(s * ROWS_PER_TILE, ROWS_PER_TILE)])
    pltpu.sync_copy(ones_hbm, ones_v)
    pltpu.sync_copy(dstc_hbm.at[pl.ds(wid * NCHUNK, NCHUNK)], slab_v)
    plsc.subcore_barrier()

    def body(j, carry):
        pltpu.async_copy(ones_v, acc_sh.at[slab_v.at[j]], sem, add=True)
        return carry

    lax.fori_loop(0, NCHUNK, body, 0)

    def drain(j, carry):
        pltpu.make_async_copy(ones_v, acc_sh.at[slab_v.at[0]], sem).wait()
        return carry

    lax.fori_loop(0, NCHUNK, drain, 0)
    plsc.subcore_barrier()
    r0 = s * ROWS_PER_TILE
    pltpu.sync_copy(acc_sh.at[pl.ds(r0, ROWS_PER_TILE)], zbuf_v)
    pltpu.sync_copy(zbuf_v, deg_out.at[pl.ds(c * N_PAD + r0, ROWS_PER_TILE)])


def _unpack_chunk(slab_v, j, src_v, dst_v):
    for i in range(CH // 16):
        v = slab_v[j, pl.ds(i * 16, 16)]
        src_v[pl.ds(i * 16, 16)] = lax.bitwise_and(v, 0xFFFF)
        dst_v[pl.ds(i * 16, 16)] = lax.shift_right_logical(v, 16)


@functools.partial(
    pl.kernel,
    out_type=jax.ShapeDtypeStruct((NC, N_PAD, D), jnp.float32),
    mesh=_mesh,
    scratch_types=[
        pltpu.VMEM_SHARED((N_PAD, D), jnp.float32),
        pltpu.VMEM((NCHUNK, CH), jnp.int32),
        [pltpu.VMEM((CH, D), jnp.float32) for _ in range(NBUF)],
        [pltpu.VMEM((CH,), jnp.int32) for _ in range(NBUF)],
        [pltpu.VMEM((CH,), jnp.int32) for _ in range(NBUF)],
        [pltpu.SemaphoreType.DMA for _ in range(NBUF)],
        [pltpu.SemaphoreType.DMA for _ in range(NBUF)],
    ],
)
def _sc_aggregate(xws_hbm, sdt_hbm, zrows_hbm, agg_out, acc_sh, slab_v,
                  rows_vs, srcs_vs, dsts_vs, gsems, wsems):
    c = lax.axis_index("c")
    s = lax.axis_index("s")
    wid = c * NS + s
    pltpu.sync_copy(zrows_hbm, rows_vs[0])
    r0 = s * ROWS_PER_TILE
    for k, sz in enumerate(DUMP_SIZES):
        pltpu.async_copy(rows_vs[0].at[pl.ds(0, sz), :],
                         acc_sh.at[pl.ds(r0 + 128 * k, sz), :], wsems[0])
    pltpu.sync_copy(sdt_hbm.at[pl.ds(wid * NCHUNK, NCHUNK)], slab_v)
    for k, sz in enumerate(DUMP_SIZES):
        pltpu.make_async_copy(rows_vs[0].at[pl.ds(0, sz), :],
                              acc_sh.at[pl.ds(r0 + 128 * k, sz), :],
                              wsems[0]).wait()
    plsc.subcore_barrier()

    for b in range(NBUF):
        _unpack_chunk(slab_v, b, srcs_vs[b], dsts_vs[b])
        pltpu.async_copy(xws_hbm.at[srcs_vs[b]], rows_vs[b], gsems[b])

    def grp(g, carry):
        for b in range(NBUF):
            j = g * NBUF + b
            pltpu.make_async_copy(
                xws_hbm.at[srcs_vs[b]], rows_vs[b], gsems[b]).wait()
            pltpu.sync_copy(rows_vs[b], acc_sh.at[dsts_vs[b]], add=True)

            @pl.when(j + NBUF < NCHUNK)
            def _():
                _unpack_chunk(slab_v, j + NBUF, srcs_vs[b], dsts_vs[b])
                pltpu.async_copy(xws_hbm.at[srcs_vs[b]], rows_vs[b], gsems[b])
        return carry

    lax.fori_loop(0, NCHUNK // NBUF, grp, 0)
    plsc.subcore_barrier()
    for k, sz in enumerate(DUMP_SIZES):
        b = k % NBUF
        if k >= NBUF:
            szp = DUMP_SIZES[k - NBUF]
            pltpu.make_async_copy(
                rows_vs[b].at[pl.ds(0, szp), :],
                agg_out.at[c, pl.ds(r0 + 128 * (k - NBUF), szp), :],
                wsems[b]).wait()
        pltpu.sync_copy(acc_sh.at[pl.ds(r0 + 128 * k, sz), :],
                        rows_vs[b].at[pl.ds(0, sz), :])
        pltpu.async_copy(rows_vs[b].at[pl.ds(0, sz), :],
                         agg_out.at[c, pl.ds(r0 + 128 * k, sz), :], wsems[b])
    for k in range(len(DUMP_SIZES) - NBUF, len(DUMP_SIZES)):
        b = k % NBUF
        sz = DUMP_SIZES[k]
        pltpu.make_async_copy(rows_vs[b].at[pl.ds(0, sz), :],
                              agg_out.at[c, pl.ds(r0 + 128 * k, sz), :],
                              wsems[b]).wait()


def _tc_mm_body(t_ref, z_ref, wt1_ref, bt1_ref, wt2_ref, bt2_ref,
                wenc_ref, xw1_ref, temb_ref):
    t_in = t_ref[...].astype(jnp.float32)
    h = t_in * wt1_ref[...] + bt1_ref[...]
    h = 0.5 * h * (1.0 + lax.erf(h * 0.7071067811865476))
    temb_ref[...] = jnp.dot(h, wt2_ref[...],
                            preferred_element_type=jnp.float32) + bt2_ref[...]
    xw1_ref[...] = jnp.dot(z_ref[...], wenc_ref[...],
                           preferred_element_type=jnp.float32)


def _tc_scale_body(degp_ref, xw1_ref, xws1_ref, dinv_ref):
    deg = degp_ref[0, :] + degp_ref[1, :] + 1.0
    dinv_all = lax.rsqrt(deg)
    dinv = dinv_all[:N].reshape(N, 1)
    dinv_ref[...] = dinv
    xws1_ref[:N, :] = xw1_ref[...] * dinv
    xws1_ref[N:, :] = jnp.zeros((N_PAD - N, D), jnp.float32)


def _tc_mid_body(p_ref, xws1_ref, dinv_ref, temb_ref, benc_ref, wdec_ref,
                 xws2_ref):
    dinv = dinv_ref[...]
    agg = p_ref[0, :N, :] + p_ref[1, :N, :] + xws1_ref[:N, :]
    pre = dinv * agg + benc_ref[...] + temb_ref[...]
    h1 = jnp.where(pre > 0, pre, jnp.exp(jnp.minimum(pre, 0.0)) - 1.0)
    xw2 = jnp.dot(h1, wdec_ref[...], preferred_element_type=jnp.float32)
    xws2_ref[:N, :] = xw2 * dinv
    xws2_ref[N:, :] = jnp.zeros((N_PAD - N, D), jnp.float32)


def _tc_post_body(q_ref, xws2_ref, dinv_ref, bdec_ref, out_ref):
    agg = q_ref[0, :N, :] + q_ref[1, :N, :] + xws2_ref[:N, :]
    out_ref[...] = dinv_ref[...] * agg + bdec_ref[...]


def kernel(z, edge_index, t, W_t1, b_t1, W_t2, b_t2, W_enc, b_enc, W_dec,
           b_dec):
    src = edge_index[0].astype(jnp.int32)
    dst = edge_index[1].astype(jnp.int32)
    slop_idx = N + (jnp.arange(EP - E, dtype=jnp.int32) % NSLOP)
    srcp = jnp.concatenate([src, slop_idx])
    dstp = jnp.concatenate([dst, slop_idx])
    sdt = (srcp | (dstp << 16)).reshape(-1, CH)
    dstc = dstp.reshape(-1, CH)

    ones_ch = jnp.ones((CH,), jnp.float32)
    zvec = jnp.zeros((ROWS_PER_TILE,), jnp.float32)
    zrows = jnp.zeros((CH, D), jnp.float32)

    degp = _sc_degree(dstc, ones_ch, zvec).reshape(NC, N_PAD)

    xw1, temb = pl.pallas_call(
        _tc_mm_body,
        out_shape=(
            jax.ShapeDtypeStruct((N, D), jnp.float32),
            jax.ShapeDtypeStruct((N, D), jnp.float32),
        ),
    )(t.astype(jnp.int32).reshape(N, 1), z, W_t1, b_t1.reshape(1, D),
      W_t2, b_t2.reshape(1, D), W_enc)

    xws1, dinv = pl.pallas_call(
        _tc_scale_body,
        out_shape=(
            jax.ShapeDtypeStruct((N_PAD, D), jnp.float32),
            jax.ShapeDtypeStruct((N, 1), jnp.float32),
        ),
    )(degp, xw1)

    p = _sc_aggregate(xws1, sdt, zrows)

    xws2 = pl.pallas_call(
        _tc_mid_body,
        out_shape=jax.ShapeDtypeStruct((N_PAD, D), jnp.float32),
    )(p, xws1, dinv, temb, b_enc.reshape(1, D), W_dec)

    q = _sc_aggregate(xws2, sdt, zrows)

    out = pl.pallas_call(
        _tc_post_body,
        out_shape=jax.ShapeDtypeStruct((N, D), jnp.float32),
    )(q, xws2, dinv, b_dec.reshape(1, D))
    return out


# speedup vs baseline: 1.0110x; 1.0110x over previous
import functools
import jax
import jax.numpy as jnp
from jax import lax
from jax.experimental import pallas as pl
from jax.experimental.pallas import tpu as pltpu
from jax.experimental.pallas import tpu_sc as plsc

N = 10000
E = 320000
D = 128

NC = 2
NS = 16
NW = NC * NS

CH = 128
NCHUNK = 80
EP = NW * NCHUNK * CH
NBUF = 2
N_PAD = 10112
ROWS_PER_TILE = N_PAD // NS
DUMP_SIZES = (128, 128, 128, 128, 120)
NSLOP = N_PAD - N

_mesh = plsc.VectorSubcoreMesh(core_axis_name="c", subcore_axis_name="s",
                               num_cores=NC, num_subcores=NS)


@functools.partial(
    pl.kernel,
    out_type=jax.ShapeDtypeStruct((NC * N_PAD,), jnp.float32),
    mesh=_mesh,
    scratch_types=[
        pltpu.VMEM_SHARED((N_PAD,), jnp.float32),
        pltpu.VMEM((NCHUNK, CH), jnp.int32),
        pltpu.VMEM((CH,), jnp.float32),
        pltpu.VMEM((ROWS_PER_TILE,), jnp.float32),
        pltpu.SemaphoreType.DMA,
    ],
)
def _sc_degree(dstc_hbm, ones_hbm, zvec_hbm, deg_out, acc_sh, slab_v, ones_v,
               zbuf_v, sem):
    c = lax.axis_index("c")
    s = lax.axis_index("s")
    wid = c * NS + s
    pltpu.sync_copy(zvec_hbm, zbuf_v)
    pltpu.sync_copy(zbuf_v, acc_sh.at[pl.ds(s * ROWS_PER_TILE, ROWS_PER_TILE)])
    pltpu.sync_copy(ones_hbm, ones_v)
    pltpu.sync_copy(dstc_hbm.at[pl.ds(wid * NCHUNK, NCHUNK)], slab_v)
    plsc.subcore_barrier()

    def body(j, carry):
        pltpu.async_copy(ones_v, acc_sh.at[slab_v.at[j]], sem, add=True)
        return carry

    lax.fori_loop(0, NCHUNK, body, 0)

    def drain(j, carry):
        pltpu.make_async_copy(ones_v, acc_sh.at[slab_v.at[0]], sem).wait()
        return carry

    lax.fori_loop(0, NCHUNK, drain, 0)
    plsc.subcore_barrier()
    r0 = s * ROWS_PER_TILE
    pltpu.sync_copy(acc_sh.at[pl.ds(r0, ROWS_PER_TILE)], zbuf_v)
    pltpu.sync_copy(zbuf_v, deg_out.at[pl.ds(c * N_PAD + r0, ROWS_PER_TILE)])


def _unpack_chunk(slab_v, j, src_v, dst_v):
    for i in range(CH // 16):
        v = slab_v[j, pl.ds(i * 16, 16)]
        src_v[pl.ds(i * 16, 16)] = lax.bitwise_and(v, 0xFFFF)
        dst_v[pl.ds(i * 16, 16)] = lax.shift_right_logical(v, 16)


@functools.partial(
    pl.kernel,
    out_type=jax.ShapeDtypeStruct((NC, N_PAD, D), jnp.float32),
    mesh=_mesh,
    scratch_types=[
        pltpu.VMEM_SHARED((N_PAD, D), jnp.float32),
        pltpu.VMEM((NCHUNK, CH), jnp.int32),
        [pltpu.VMEM((CH, D), jnp.float32) for _ in range(NBUF)],
        [pltpu.VMEM((CH,), jnp.int32) for _ in range(NBUF)],
        [pltpu.VMEM((CH,), jnp.int32) for _ in range(NBUF)],
        [pltpu.SemaphoreType.DMA for _ in range(NBUF)],
        [pltpu.SemaphoreType.DMA for _ in range(NBUF)],
    ],
)
def _sc_aggregate(xws_hbm, sdt_hbm, zrows_hbm, agg_out, acc_sh, slab_v,
                  rows_vs, srcs_vs, dsts_vs, gsems, wsems):
    c = lax.axis_index("c")
    s = lax.axis_index("s")
    wid = c * NS + s
    pltpu.sync_copy(zrows_hbm, rows_vs[0])
    r0 = s * ROWS_PER_TILE
    for k, sz in enumerate(DUMP_SIZES):
        pltpu.async_copy(rows_vs[0].at[pl.ds(0, sz), :],
                         acc_sh.at[pl.ds(r0 + 128 * k, sz), :], wsems[0])
    pltpu.sync_copy(sdt_hbm.at[pl.ds(wid * NCHUNK, NCHUNK)], slab_v)
    for k, sz in enumerate(DUMP_SIZES):
        pltpu.make_async_copy(rows_vs[0].at[pl.ds(0, sz), :],
                              acc_sh.at[pl.ds(r0 + 128 * k, sz), :],
                              wsems[0]).wait()
    plsc.subcore_barrier()

    for b in range(NBUF):
        _unpack_chunk(slab_v, b, srcs_vs[b], dsts_vs[b])
        pltpu.async_copy(xws_hbm.at[srcs_vs[b]], rows_vs[b], gsems[b])

    def grp(g, carry):
        for b in range(NBUF):
            j = g * NBUF + b
            pltpu.make_async_copy(
                xws_hbm.at[srcs_vs[b]], rows_vs[b], gsems[b]).wait()
            pltpu.sync_copy(rows_vs[b], acc_sh.at[dsts_vs[b]], add=True)

            @pl.when(j + NBUF < NCHUNK)
            def _():
                _unpack_chunk(slab_v, j + NBUF, srcs_vs[b], dsts_vs[b])
                pltpu.async_copy(xws_hbm.at[srcs_vs[b]], rows_vs[b], gsems[b])
        return carry

    lax.fori_loop(0, NCHUNK // NBUF, grp, 0)
    plsc.subcore_barrier()
    for k, sz in enumerate(DUMP_SIZES):
        b = k % NBUF
        if k >= NBUF:
            szp = DUMP_SIZES[k - NBUF]
            pltpu.make_async_copy(
                rows_vs[b].at[pl.ds(0, szp), :],
                agg_out.at[c, pl.ds(r0 + 128 * (k - NBUF), szp), :],
                wsems[b]).wait()
        pltpu.sync_copy(acc_sh.at[pl.ds(r0 + 128 * k, sz), :],
                        rows_vs[b].at[pl.ds(0, sz), :])
        pltpu.async_copy(rows_vs[b].at[pl.ds(0, sz), :],
                         agg_out.at[c, pl.ds(r0 + 128 * k, sz), :], wsems[b])
    for k in range(len(DUMP_SIZES) - NBUF, len(DUMP_SIZES)):
        b = k % NBUF
        sz = DUMP_SIZES[k]
        pltpu.make_async_copy(rows_vs[b].at[pl.ds(0, sz), :],
                              agg_out.at[c, pl.ds(r0 + 128 * k, sz), :],
                              wsems[b]).wait()


def _tc_pre_body(degp_ref, t_ref, z_ref, wt1_ref, bt1_ref, wt2_ref, bt2_ref,
                 wenc_ref, xws1_ref, dinv_ref, temb_ref):
    deg = degp_ref[0, :] + degp_ref[1, :] + 1.0
    dinv_all = lax.rsqrt(deg)
    dinv = dinv_all[:N].reshape(N, 1)
    dinv_ref[...] = dinv
    t_in = t_ref[...].astype(jnp.float32)
    h = t_in * wt1_ref[...] + bt1_ref[...]
    h = 0.5 * h * (1.0 + lax.erf(h * 0.7071067811865476))
    temb_ref[...] = jnp.dot(h, wt2_ref[...],
                            preferred_element_type=jnp.float32) + bt2_ref[...]
    xw1 = jnp.dot(z_ref[...], wenc_ref[...],
                  preferred_element_type=jnp.float32)
    xws1_ref[:N, :] = xw1 * dinv
    xws1_ref[N:, :] = jnp.zeros((N_PAD - N, D), jnp.float32)


def _tc_mid_body(p_ref, xws1_ref, dinv_ref, temb_ref, benc_ref, wdec_ref,
                 xws2_ref):
    dinv = dinv_ref[...]
    agg = p_ref[0, :N, :] + p_ref[1, :N, :] + xws1_ref[:N, :]
    pre = dinv * agg + benc_ref[...] + temb_ref[...]
    h1 = jnp.where(pre > 0, pre, jnp.exp(jnp.minimum(pre, 0.0)) - 1.0)
    xw2 = jnp.dot(h1, wdec_ref[...], preferred_element_type=jnp.float32)
    xws2_ref[:N, :] = xw2 * dinv
    xws2_ref[N:, :] = jnp.zeros((N_PAD - N, D), jnp.float32)


def _tc_post_body(q_ref, xws2_ref, dinv_ref, bdec_ref, out_ref):
    agg = q_ref[0, :N, :] + q_ref[1, :N, :] + xws2_ref[:N, :]
    out_ref[...] = dinv_ref[...] * agg + bdec_ref[...]


def kernel(z, edge_index, t, W_t1, b_t1, W_t2, b_t2, W_enc, b_enc, W_dec,
           b_dec):
    src = edge_index[0].astype(jnp.int32)
    dst = edge_index[1].astype(jnp.int32)
    slop_idx = N + (jnp.arange(EP - E, dtype=jnp.int32) % NSLOP)
    srcp = jnp.concatenate([src, slop_idx])
    dstp = jnp.concatenate([dst, slop_idx])
    sdt = (srcp | (dstp << 16)).reshape(-1, CH)
    dstc = dstp.reshape(-1, CH)

    ones_ch = jnp.ones((CH,), jnp.float32)
    zvec = jnp.zeros((ROWS_PER_TILE,), jnp.float32)
    zrows = jnp.zeros((CH, D), jnp.float32)

    degp = _sc_degree(dstc, ones_ch, zvec).reshape(NC, N_PAD)

    xws1, dinv, temb = pl.pallas_call(
        _tc_pre_body,
        out_shape=(
            jax.ShapeDtypeStruct((N_PAD, D), jnp.float32),
            jax.ShapeDtypeStruct((N, 1), jnp.float32),
            jax.ShapeDtypeStruct((N, D), jnp.float32),
        ),
    )(degp, t.astype(jnp.int32).reshape(N, 1), z, W_t1, b_t1.reshape(1, D),
      W_t2, b_t2.reshape(1, D), W_enc)

    p = _sc_aggregate(xws1, sdt, zrows)

    xws2 = pl.pallas_call(
        _tc_mid_body,
        out_shape=jax.ShapeDtypeStruct((N_PAD, D), jnp.float32),
    )(p, xws1, dinv, temb, b_enc.reshape(1, D), W_dec)

    q = _sc_aggregate(xws2, sdt, zrows)

    out = pl.pallas_call(
        _tc_post_body,
        out_shape=jax.ShapeDtypeStruct((N, D), jnp.float32),
    )(q, xws2, dinv, b_dec.reshape(1, D))
    return out


# speedup vs baseline: 1.0323x; 1.0210x over previous
import functools
import jax
import jax.numpy as jnp
from jax import lax
from jax.experimental import pallas as pl
from jax.experimental.pallas import tpu as pltpu
from jax.experimental.pallas import tpu_sc as plsc

N = 10000
E = 320000
D = 128

NC = 2
NS = 16
NW = NC * NS

CH = 128
NCHUNK = 80
EP = NW * NCHUNK * CH
NBUF = 2
N_PAD = 10112
ROWS_PER_TILE = N_PAD // NS
DUMP_SIZES = (128, 128, 128, 128, 120)
NSLOP = N_PAD - N

_mesh = plsc.VectorSubcoreMesh(core_axis_name="c", subcore_axis_name="s",
                               num_cores=NC, num_subcores=NS)


@functools.partial(
    pl.kernel,
    out_type=jax.ShapeDtypeStruct((NC * N_PAD,), jnp.float32),
    mesh=_mesh,
    scratch_types=[
        pltpu.VMEM_SHARED((N_PAD,), jnp.float32),
        pltpu.VMEM((NCHUNK, CH), jnp.int32),
        pltpu.VMEM((CH,), jnp.float32),
        pltpu.VMEM((ROWS_PER_TILE,), jnp.float32),
        pltpu.SemaphoreType.DMA,
    ],
)
def _sc_degree(dstc_hbm, ones_hbm, zvec_hbm, deg_out, acc_sh, slab_v, ones_v,
               zbuf_v, sem):
    c = lax.axis_index("c")
    s = lax.axis_index("s")
    wid = c * NS + s
    pltpu.sync_copy(zvec_hbm, zbuf_v)
    pltpu.sync_copy(zbuf_v, acc_sh.at[pl.ds(s * ROWS_PER_TILE, ROWS_PER_TILE)])
    pltpu.sync_copy(ones_hbm, ones_v)
    pltpu.sync_copy(dstc_hbm.at[pl.ds(wid * NCHUNK, NCHUNK)], slab_v)
    plsc.subcore_barrier()

    def body(j, carry):
        pltpu.async_copy(ones_v, acc_sh.at[slab_v.at[j]], sem, add=True)
        return carry

    lax.fori_loop(0, NCHUNK, body, 0)

    def drain(j, carry):
        pltpu.make_async_copy(ones_v, acc_sh.at[slab_v.at[0]], sem).wait()
        return carry

    lax.fori_loop(0, NCHUNK, drain, 0)
    plsc.subcore_barrier()
    r0 = s * ROWS_PER_TILE
    pltpu.sync_copy(acc_sh.at[pl.ds(r0, ROWS_PER_TILE)], zbuf_v)
    pltpu.sync_copy(zbuf_v, deg_out.at[pl.ds(c * N_PAD + r0, ROWS_PER_TILE)])


def _unpack_chunk(slab_v, j, src_v, dst_v):
    for i in range(CH // 16):
        v = slab_v[j, pl.ds(i * 16, 16)]
        src_v[pl.ds(i * 16, 16)] = lax.bitwise_and(v, 0xFFFF)
        dst_v[pl.ds(i * 16, 16)] = lax.shift_right_logical(v, 16)


@functools.partial(
    pl.kernel,
    out_type=jax.ShapeDtypeStruct((NC, N_PAD, D), jnp.float32),
    mesh=_mesh,
    scratch_types=[
        pltpu.VMEM_SHARED((N_PAD, D), jnp.float32),
        pltpu.VMEM((NCHUNK, CH), jnp.int32),
        [pltpu.VMEM((CH, D), jnp.float32) for _ in range(NBUF)],
        [pltpu.VMEM((CH,), jnp.int32) for _ in range(NBUF)],
        [pltpu.VMEM((CH,), jnp.int32) for _ in range(NBUF)],
        [pltpu.SemaphoreType.DMA for _ in range(NBUF)],
        [pltpu.SemaphoreType.DMA for _ in range(NBUF)],
    ],
)
def _sc_aggregate(xws_hbm, sdt_hbm, zrows_hbm, agg_out, acc_sh, slab_v,
                  rows_vs, srcs_vs, dsts_vs, gsems, wsems):
    c = lax.axis_index("c")
    s = lax.axis_index("s")
    wid = c * NS + s
    pltpu.sync_copy(zrows_hbm, rows_vs[0])
    r0 = s * ROWS_PER_TILE
    for k, sz in enumerate(DUMP_SIZES):
        pltpu.async_copy(rows_vs[0].at[pl.ds(0, sz), :],
                         acc_sh.at[pl.ds(r0 + 128 * k, sz), :], wsems[0])
    pltpu.sync_copy(sdt_hbm.at[pl.ds(wid * NCHUNK, NCHUNK)], slab_v)
    for k, sz in enumerate(DUMP_SIZES):
        pltpu.make_async_copy(rows_vs[0].at[pl.ds(0, sz), :],
                              acc_sh.at[pl.ds(r0 + 128 * k, sz), :],
                              wsems[0]).wait()
    plsc.subcore_barrier()

    for b in range(NBUF):
        _unpack_chunk(slab_v, b, srcs_vs[b], dsts_vs[b])
        pltpu.async_copy(xws_hbm.at[srcs_vs[b]], rows_vs[b], gsems[b])

    def grp(g, carry):
        for b in range(NBUF):
            j = g * NBUF + b
            pltpu.make_async_copy(
                xws_hbm.at[srcs_vs[b]], rows_vs[b], gsems[b]).wait()
            pltpu.sync_copy(rows_vs[b], acc_sh.at[dsts_vs[b]], add=True)

            @pl.when(j + NBUF < NCHUNK)
            def _():
                _unpack_chunk(slab_v, j + NBUF, srcs_vs[b], dsts_vs[b])
                pltpu.async_copy(xws_hbm.at[srcs_vs[b]], rows_vs[b], gsems[b])
        return carry

    lax.fori_loop(0, NCHUNK // NBUF, grp, 0)
    plsc.subcore_barrier()
    for k, sz in enumerate(DUMP_SIZES):
        b = k % NBUF
        if k >= NBUF:
            szp = DUMP_SIZES[k - NBUF]
            pltpu.make_async_copy(
                rows_vs[b].at[pl.ds(0, szp), :],
                agg_out.at[c, pl.ds(r0 + 128 * (k - NBUF), szp), :],
                wsems[b]).wait()
        pltpu.sync_copy(acc_sh.at[pl.ds(r0 + 128 * k, sz), :],
                        rows_vs[b].at[pl.ds(0, sz), :])
        pltpu.async_copy(rows_vs[b].at[pl.ds(0, sz), :],
                         agg_out.at[c, pl.ds(r0 + 128 * k, sz), :], wsems[b])
    for k in range(len(DUMP_SIZES) - NBUF, len(DUMP_SIZES)):
        b = k % NBUF
        sz = DUMP_SIZES[k]
        pltpu.make_async_copy(rows_vs[b].at[pl.ds(0, sz), :],
                              agg_out.at[c, pl.ds(r0 + 128 * k, sz), :],
                              wsems[b]).wait()


def _tc_pack_body(ei_ref, sdt_ref, dstc_ref):
    src3 = ei_ref[0, :, :]
    dst3 = ei_ref[1, :, :]
    sdt_ref[:E // CH, :] = src3 | lax.shift_left(dst3, 16)
    dstc_ref[:E // CH, :] = dst3
    nsl = EP // CH - E // CH
    row = lax.broadcasted_iota(jnp.int32, (nsl, CH), 0)
    col = lax.broadcasted_iota(jnp.int32, (nsl, CH), 1)
    slop = N + lax.rem(row * CH + col, NSLOP)
    sdt_ref[E // CH:, :] = slop | lax.shift_left(slop, 16)
    dstc_ref[E // CH:, :] = slop


def _tc_pre_body(degp_ref, t_ref, z_ref, wt1_ref, bt1_ref, wt2_ref, bt2_ref,
                 wenc_ref, xws1_ref, dinv_ref, temb_ref):
    deg = degp_ref[0, :] + degp_ref[1, :] + 1.0
    dinv_all = lax.rsqrt(deg)
    dinv = dinv_all[:N].reshape(N, 1)
    dinv_ref[...] = dinv
    t_in = t_ref[...].astype(jnp.float32)
    h = t_in * wt1_ref[...] + bt1_ref[...]
    h = 0.5 * h * (1.0 + lax.erf(h * 0.7071067811865476))
    temb_ref[...] = jnp.dot(h, wt2_ref[...],
                            preferred_element_type=jnp.float32) + bt2_ref[...]
    xw1 = jnp.dot(z_ref[...], wenc_ref[...],
                  preferred_element_type=jnp.float32)
    xws1_ref[:N, :] = xw1 * dinv
    xws1_ref[N:, :] = jnp.zeros((N_PAD - N, D), jnp.float32)


def _tc_mid_body(p_ref, xws1_ref, dinv_ref, temb_ref, benc_ref, wdec_ref,
                 xws2_ref):
    dinv = dinv_ref[...]
    agg = p_ref[0, :N, :] + p_ref[1, :N, :] + xws1_ref[:N, :]
    pre = dinv * agg + benc_ref[...] + temb_ref[...]
    h1 = jnp.where(pre > 0, pre, jnp.exp(jnp.minimum(pre, 0.0)) - 1.0)
    xw2 = jnp.dot(h1, wdec_ref[...], preferred_element_type=jnp.float32)
    xws2_ref[:N, :] = xw2 * dinv
    xws2_ref[N:, :] = jnp.zeros((N_PAD - N, D), jnp.float32)


def _tc_post_body(q_ref, xws2_ref, dinv_ref, bdec_ref, out_ref):
    agg = q_ref[0, :N, :] + q_ref[1, :N, :] + xws2_ref[:N, :]
    out_ref[...] = dinv_ref[...] * agg + bdec_ref[...]


def kernel(z, edge_index, t, W_t1, b_t1, W_t2, b_t2, W_enc, b_enc, W_dec,
           b_dec):
    ei3 = edge_index.astype(jnp.int32).reshape(2, E // CH, CH)
    sdt, dstc = pl.pallas_call(
        _tc_pack_body,
        out_shape=(
            jax.ShapeDtypeStruct((EP // CH, CH), jnp.int32),
            jax.ShapeDtypeStruct((EP // CH, CH), jnp.int32),
        ),
    )(ei3)

    ones_ch = jnp.ones((CH,), jnp.float32)
    zvec = jnp.zeros((ROWS_PER_TILE,), jnp.float32)
    zrows = jnp.zeros((CH, D), jnp.float32)

    degp = _sc_degree(dstc, ones_ch, zvec).reshape(NC, N_PAD)

    xws1, dinv, temb = pl.pallas_call(
        _tc_pre_body,
        out_shape=(
            jax.ShapeDtypeStruct((N_PAD, D), jnp.float32),
            jax.ShapeDtypeStruct((N, 1), jnp.float32),
            jax.ShapeDtypeStruct((N, D), jnp.float32),
        ),
    )(degp, t.astype(jnp.int32).reshape(N, 1), z, W_t1, b_t1.reshape(1, D),
      W_t2, b_t2.reshape(1, D), W_enc)

    p = _sc_aggregate(xws1, sdt, zrows)

    xws2 = pl.pallas_call(
        _tc_mid_body,
        out_shape=jax.ShapeDtypeStruct((N_PAD, D), jnp.float32),
    )(p, xws1, dinv, temb, b_enc.reshape(1, D), W_dec)

    q = _sc_aggregate(xws2, sdt, zrows)

    out = pl.pallas_call(
        _tc_post_body,
        out_shape=jax.ShapeDtypeStruct((N, D), jnp.float32),
    )(q, xws2, dinv, b_dec.reshape(1, D))
    return out


# speedup vs baseline: 1.0581x; 1.0251x over previous
import functools
import jax
import jax.numpy as jnp
from jax import lax
from jax.experimental import pallas as pl
from jax.experimental.pallas import tpu as pltpu
from jax.experimental.pallas import tpu_sc as plsc

N = 10000
E = 320000
D = 128

NC = 2
NS = 16
NW = NC * NS

CH = 128
NCHUNK = 80
EP = NW * NCHUNK * CH
NBUF = 2
N_PAD = 10112
ROWS_PER_TILE = N_PAD // NS
DUMP_SIZES = (128, 128, 128, 128, 120)
NSLOP = N_PAD - N

_mesh = plsc.VectorSubcoreMesh(core_axis_name="c", subcore_axis_name="s",
                               num_cores=NC, num_subcores=NS)


@functools.partial(
    pl.kernel,
    out_type=jax.ShapeDtypeStruct((NC * N_PAD,), jnp.float32),
    mesh=_mesh,
    scratch_types=[
        pltpu.VMEM_SHARED((N_PAD,), jnp.float32),
        pltpu.VMEM((NCHUNK, CH), jnp.int32),
        pltpu.VMEM((CH,), jnp.float32),
        pltpu.VMEM((ROWS_PER_TILE,), jnp.float32),
        pltpu.SemaphoreType.DMA,
    ],
)
def _sc_degree(dstc_hbm, ones_hbm, zvec_hbm, deg_out, acc_sh, slab_v, ones_v,
               zbuf_v, sem):
    c = lax.axis_index("c")
    s = lax.axis_index("s")
    wid = c * NS + s
    pltpu.sync_copy(zvec_hbm, zbuf_v)
    pltpu.sync_copy(zbuf_v, acc_sh.at[pl.ds(s * ROWS_PER_TILE, ROWS_PER_TILE)])
    pltpu.sync_copy(ones_hbm, ones_v)
    pltpu.sync_copy(dstc_hbm.at[pl.ds(wid * NCHUNK, NCHUNK)], slab_v)
    plsc.subcore_barrier()

    def body(j, carry):
        pltpu.async_copy(ones_v, acc_sh.at[slab_v.at[j]], sem, add=True)
        return carry

    lax.fori_loop(0, NCHUNK, body, 0)

    def drain(j, carry):
        pltpu.make_async_copy(ones_v, acc_sh.at[slab_v.at[0]], sem).wait()
        return carry

    lax.fori_loop(0, NCHUNK, drain, 0)
    plsc.subcore_barrier()
    r0 = s * ROWS_PER_TILE
    pltpu.sync_copy(acc_sh.at[pl.ds(r0, ROWS_PER_TILE)], zbuf_v)
    pltpu.sync_copy(zbuf_v, deg_out.at[pl.ds(c * N_PAD + r0, ROWS_PER_TILE)])


def _unpack_chunk(slab_v, j, src_v, dst_v):
    for i in range(CH // 16):
        v = slab_v[j, pl.ds(i * 16, 16)]
        src_v[pl.ds(i * 16, 16)] = lax.bitwise_and(v, 0xFFFF)
        dst_v[pl.ds(i * 16, 16)] = lax.shift_right_logical(v, 16)


@functools.partial(
    pl.kernel,
    out_type=jax.ShapeDtypeStruct((NC, N_PAD, D), jnp.float32),
    mesh=_mesh,
    scratch_types=[
        pltpu.VMEM_SHARED((N_PAD, D), jnp.float32),
        pltpu.VMEM((NCHUNK, CH), jnp.int32),
        [pltpu.VMEM((CH, D), jnp.float32) for _ in range(NBUF)],
        [pltpu.VMEM((CH,), jnp.int32) for _ in range(NBUF)],
        [pltpu.VMEM((CH,), jnp.int32) for _ in range(NBUF)],
        [pltpu.SemaphoreType.DMA for _ in range(NBUF)],
        [pltpu.SemaphoreType.DMA for _ in range(NBUF)],
    ],
)
def _sc_aggregate(xws_hbm, sdt_hbm, zrows_hbm, agg_out, acc_sh, slab_v,
                  rows_vs, srcs_vs, dsts_vs, gsems, wsems):
    c = lax.axis_index("c")
    s = lax.axis_index("s")
    wid = c * NS + s
    pltpu.sync_copy(zrows_hbm, rows_vs[0])
    r0 = s * ROWS_PER_TILE
    for k, sz in enumerate(DUMP_SIZES):
        pltpu.async_copy(rows_vs[0].at[pl.ds(0, sz), :],
                         acc_sh.at[pl.ds(r0 + 128 * k, sz), :], wsems[0])
    pltpu.sync_copy(sdt_hbm.at[pl.ds(wid * NCHUNK, NCHUNK)], slab_v)
    for k, sz in enumerate(DUMP_SIZES):
        pltpu.make_async_copy(rows_vs[0].at[pl.ds(0, sz), :],
                              acc_sh.at[pl.ds(r0 + 128 * k, sz), :],
                              wsems[0]).wait()
    plsc.subcore_barrier()

    for b in range(NBUF):
        _unpack_chunk(slab_v, b, srcs_vs[b], dsts_vs[b])
        pltpu.async_copy(xws_hbm.at[srcs_vs[b]], rows_vs[b], gsems[b])

    def grp(g, carry):
        for b in range(NBUF):
            j = g * NBUF + b
            pltpu.make_async_copy(
                xws_hbm.at[srcs_vs[b]], rows_vs[b], gsems[b]).wait()
            pltpu.sync_copy(rows_vs[b], acc_sh.at[dsts_vs[b]], add=True)

            @pl.when(j + NBUF < NCHUNK)
            def _():
                _unpack_chunk(slab_v, j + NBUF, srcs_vs[b], dsts_vs[b])
                pltpu.async_copy(xws_hbm.at[srcs_vs[b]], rows_vs[b], gsems[b])
        return carry

    lax.fori_loop(0, NCHUNK // NBUF, grp, 0)
    plsc.subcore_barrier()
    for k, sz in enumerate(DUMP_SIZES):
        b = k % NBUF
        if k >= NBUF:
            szp = DUMP_SIZES[k - NBUF]
            pltpu.make_async_copy(
                rows_vs[b].at[pl.ds(0, szp), :],
                agg_out.at[c, pl.ds(r0 + 128 * (k - NBUF), szp), :],
                wsems[b]).wait()
        pltpu.sync_copy(acc_sh.at[pl.ds(r0 + 128 * k, sz), :],
                        rows_vs[b].at[pl.ds(0, sz), :])
        pltpu.async_copy(rows_vs[b].at[pl.ds(0, sz), :],
                         agg_out.at[c, pl.ds(r0 + 128 * k, sz), :], wsems[b])
    for k in range(len(DUMP_SIZES) - NBUF, len(DUMP_SIZES)):
        b = k % NBUF
        sz = DUMP_SIZES[k]
        pltpu.make_async_copy(rows_vs[b].at[pl.ds(0, sz), :],
                              agg_out.at[c, pl.ds(r0 + 128 * k, sz), :],
                              wsems[b]).wait()


def _tc_pack_body(ei_ref, sdt_ref, dstc_ref):
    src3 = ei_ref[0, :].reshape(E // CH, CH)
    dst3 = ei_ref[1, :].reshape(E // CH, CH)
    sdt_ref[:E // CH, :] = src3 | lax.shift_left(dst3, 16)
    dstc_ref[:E // CH, :] = dst3
    nsl = EP // CH - E // CH
    row = lax.broadcasted_iota(jnp.int32, (nsl, CH), 0)
    col = lax.broadcasted_iota(jnp.int32, (nsl, CH), 1)
    slop = N + lax.rem(row * CH + col, NSLOP)
    sdt_ref[E // CH:, :] = slop | lax.shift_left(slop, 16)
    dstc_ref[E // CH:, :] = slop


def _tc_pre_body(degp_ref, t_ref, z_ref, wt1_ref, bt1_ref, wt2_ref, bt2_ref,
                 wenc_ref, xws1_ref, dinv_ref, temb_ref):
    deg = degp_ref[0, :] + degp_ref[1, :] + 1.0
    dinv_all = lax.rsqrt(deg)
    dinv = dinv_all[:N].reshape(N, 1)
    dinv_ref[...] = dinv
    t_in = t_ref[...].astype(jnp.float32)
    h = t_in * wt1_ref[...] + bt1_ref[...]
    h = 0.5 * h * (1.0 + lax.erf(h * 0.7071067811865476))
    temb_ref[...] = jnp.dot(h, wt2_ref[...],
                            preferred_element_type=jnp.float32) + bt2_ref[...]
    xw1 = jnp.dot(z_ref[...], wenc_ref[...],
                  preferred_element_type=jnp.float32)
    xws1_ref[:N, :] = xw1 * dinv
    xws1_ref[N:, :] = jnp.zeros((N_PAD - N, D), jnp.float32)


def _tc_mid_body(p_ref, xws1_ref, dinv_ref, temb_ref, benc_ref, wdec_ref,
                 xws2_ref):
    dinv = dinv_ref[...]
    agg = p_ref[0, :N, :] + p_ref[1, :N, :] + xws1_ref[:N, :]
    pre = dinv * agg + benc_ref[...] + temb_ref[...]
    h1 = jnp.where(pre > 0, pre, jnp.exp(jnp.minimum(pre, 0.0)) - 1.0)
    xw2 = jnp.dot(h1, wdec_ref[...], preferred_element_type=jnp.float32)
    xws2_ref[:N, :] = xw2 * dinv
    xws2_ref[N:, :] = jnp.zeros((N_PAD - N, D), jnp.float32)


def _tc_post_body(q_ref, xws2_ref, dinv_ref, bdec_ref, out_ref):
    agg = q_ref[0, :N, :] + q_ref[1, :N, :] + xws2_ref[:N, :]
    out_ref[...] = dinv_ref[...] * agg + bdec_ref[...]


def kernel(z, edge_index, t, W_t1, b_t1, W_t2, b_t2, W_enc, b_enc, W_dec,
           b_dec):
    ei = (edge_index if edge_index.dtype == jnp.int32
          else edge_index.astype(jnp.int32))
    sdt, dstc = pl.pallas_call(
        _tc_pack_body,
        out_shape=(
            jax.ShapeDtypeStruct((EP // CH, CH), jnp.int32),
            jax.ShapeDtypeStruct((EP // CH, CH), jnp.int32),
        ),
    )(ei)

    ones_ch = jnp.ones((CH,), jnp.float32)
    zvec = jnp.zeros((ROWS_PER_TILE,), jnp.float32)
    zrows = jnp.zeros((CH, D), jnp.float32)

    degp = _sc_degree(dstc, ones_ch, zvec).reshape(NC, N_PAD)

    xws1, dinv, temb = pl.pallas_call(
        _tc_pre_body,
        out_shape=(
            jax.ShapeDtypeStruct((N_PAD, D), jnp.float32),
            jax.ShapeDtypeStruct((N, 1), jnp.float32),
            jax.ShapeDtypeStruct((N, D), jnp.float32),
        ),
    )(degp, t.astype(jnp.int32).reshape(N, 1), z, W_t1, b_t1.reshape(1, D),
      W_t2, b_t2.reshape(1, D), W_enc)

    p = _sc_aggregate(xws1, sdt, zrows)

    xws2 = pl.pallas_call(
        _tc_mid_body,
        out_shape=jax.ShapeDtypeStruct((N_PAD, D), jnp.float32),
    )(p, xws1, dinv, temb, b_enc.reshape(1, D), W_dec)

    q = _sc_aggregate(xws2, sdt, zrows)

    out = pl.pallas_call(
        _tc_post_body,
        out_shape=jax.ShapeDtypeStruct((N, D), jnp.float32),
    )(q, xws2, dinv, b_dec.reshape(1, D))
    return out


# speedup vs baseline: 1.0594x; 1.0012x over previous
import functools
import jax
import jax.numpy as jnp
from jax import lax
from jax.experimental import pallas as pl
from jax.experimental.pallas import tpu as pltpu
from jax.experimental.pallas import tpu_sc as plsc

N = 10000
E = 320000
D = 128

NC = 2
NS = 16
NW = NC * NS

CH = 128
NCHUNK = 80
EP = NW * NCHUNK * CH
NBUF = 2
N_PAD = 10112
ROWS_PER_TILE = N_PAD // NS
DUMP_SIZES = (128, 128, 128, 128, 120)
NSLOP = N_PAD - N

_mesh = plsc.VectorSubcoreMesh(core_axis_name="c", subcore_axis_name="s",
                               num_cores=NC, num_subcores=NS)


@functools.partial(
    pl.kernel,
    out_type=jax.ShapeDtypeStruct((NC * N_PAD,), jnp.float32),
    mesh=_mesh,
    scratch_types=[
        pltpu.VMEM_SHARED((N_PAD,), jnp.float32),
        pltpu.VMEM((NCHUNK, CH), jnp.int32),
        pltpu.VMEM((CH,), jnp.float32),
        pltpu.VMEM((ROWS_PER_TILE,), jnp.float32),
        pltpu.SemaphoreType.DMA,
    ],
)
def _sc_degree(dstc_hbm, ones_hbm, zvec_hbm, deg_out, acc_sh, slab_v, ones_v,
               zbuf_v, sem):
    c = lax.axis_index("c")
    s = lax.axis_index("s")
    wid = c * NS + s
    pltpu.sync_copy(zvec_hbm, zbuf_v)
    pltpu.sync_copy(zbuf_v, acc_sh.at[pl.ds(s * ROWS_PER_TILE, ROWS_PER_TILE)])
    pltpu.sync_copy(ones_hbm, ones_v)
    pltpu.sync_copy(dstc_hbm.at[pl.ds(wid * NCHUNK, NCHUNK)], slab_v)
    plsc.subcore_barrier()

    def body(j, carry):
        pltpu.async_copy(ones_v, acc_sh.at[slab_v.at[j]], sem, add=True)
        return carry

    lax.fori_loop(0, NCHUNK, body, 0)

    def drain(j, carry):
        pltpu.make_async_copy(ones_v, acc_sh.at[slab_v.at[0]], sem).wait()
        return carry

    lax.fori_loop(0, NCHUNK, drain, 0)
    plsc.subcore_barrier()
    r0 = s * ROWS_PER_TILE
    pltpu.sync_copy(acc_sh.at[pl.ds(r0, ROWS_PER_TILE)], zbuf_v)
    pltpu.sync_copy(zbuf_v, deg_out.at[pl.ds(c * N_PAD + r0, ROWS_PER_TILE)])


def _unpack_chunk(slab_v, j, src_v, dst_v):
    for i in range(CH // 16):
        v = slab_v[j, pl.ds(i * 16, 16)]
        src_v[pl.ds(i * 16, 16)] = lax.bitwise_and(v, 0xFFFF)
        dst_v[pl.ds(i * 16, 16)] = lax.shift_right_logical(v, 16)


@functools.partial(
    pl.kernel,
    out_type=jax.ShapeDtypeStruct((NC, N_PAD, D), jnp.float32),
    mesh=_mesh,
    scratch_types=[
        pltpu.VMEM_SHARED((N_PAD, D), jnp.float32),
        pltpu.VMEM((NCHUNK, CH), jnp.int32),
        [pltpu.VMEM((CH, D), jnp.float32) for _ in range(NBUF)],
        [pltpu.VMEM((CH,), jnp.int32) for _ in range(2 * NBUF)],
        [pltpu.VMEM((CH,), jnp.int32) for _ in range(2 * NBUF)],
        [pltpu.SemaphoreType.DMA for _ in range(NBUF)],
        [pltpu.SemaphoreType.DMA for _ in range(NBUF)],
    ],
)
def _sc_aggregate(xws_hbm, sdt_hbm, zrows_hbm, agg_out, acc_sh, slab_v,
                  rows_vs, srcs_vs, dsts_vs, gsems, wsems):
    c = lax.axis_index("c")
    s = lax.axis_index("s")
    wid = c * NS + s
    pltpu.sync_copy(zrows_hbm, rows_vs[0])
    r0 = s * ROWS_PER_TILE
    for k, sz in enumerate(DUMP_SIZES):
        pltpu.async_copy(rows_vs[0].at[pl.ds(0, sz), :],
                         acc_sh.at[pl.ds(r0 + 128 * k, sz), :], wsems[0])
    pltpu.sync_copy(sdt_hbm.at[pl.ds(wid * NCHUNK, NCHUNK)], slab_v)
    for k, sz in enumerate(DUMP_SIZES):
        pltpu.make_async_copy(rows_vs[0].at[pl.ds(0, sz), :],
                              acc_sh.at[pl.ds(r0 + 128 * k, sz), :],
                              wsems[0]).wait()
    plsc.subcore_barrier()

    for b in range(NBUF):
        _unpack_chunk(slab_v, b, srcs_vs[2 * b], dsts_vs[2 * b])
        pltpu.async_copy(xws_hbm.at[srcs_vs[2 * b]], rows_vs[b], gsems[b])
    for b in range(NBUF):
        _unpack_chunk(slab_v, NBUF + b, srcs_vs[2 * b + 1], dsts_vs[2 * b + 1])

    def grp(g, carry):
        for half in range(2):
            for b in range(NBUF):
                j = g * 2 * NBUF + half * NBUF + b
                pltpu.make_async_copy(
                    xws_hbm.at[srcs_vs[2 * b + half]], rows_vs[b],
                    gsems[b]).wait()
                pltpu.sync_copy(rows_vs[b], acc_sh.at[dsts_vs[2 * b + half]],
                                add=True)

                @pl.when(j + NBUF < NCHUNK)
                def _():
                    pltpu.async_copy(
                        xws_hbm.at[srcs_vs[2 * b + 1 - half]], rows_vs[b],
                        gsems[b])

                @pl.when(j + 2 * NBUF < NCHUNK)
                def _():
                    _unpack_chunk(slab_v, j + 2 * NBUF, srcs_vs[2 * b + half],
                                  dsts_vs[2 * b + half])
        return carry

    lax.fori_loop(0, NCHUNK // (2 * NBUF), grp, 0)
    plsc.subcore_barrier()
    for k, sz in enumerate(DUMP_SIZES):
        b = k % NBUF
        if k >= NBUF:
            szp = DUMP_SIZES[k - NBUF]
            pltpu.make_async_copy(
                rows_vs[b].at[pl.ds(0, szp), :],
                agg_out.at[c, pl.ds(r0 + 128 * (k - NBUF), szp), :],
                wsems[b]).wait()
        pltpu.sync_copy(acc_sh.at[pl.ds(r0 + 128 * k, sz), :],
                        rows_vs[b].at[pl.ds(0, sz), :])
        pltpu.async_copy(rows_vs[b].at[pl.ds(0, sz), :],
                         agg_out.at[c, pl.ds(r0 + 128 * k, sz), :], wsems[b])
    for k in range(len(DUMP_SIZES) - NBUF, len(DUMP_SIZES)):
        b = k % NBUF
        sz = DUMP_SIZES[k]
        pltpu.make_async_copy(rows_vs[b].at[pl.ds(0, sz), :],
                              agg_out.at[c, pl.ds(r0 + 128 * k, sz), :],
                              wsems[b]).wait()


def _tc_pack_body(ei_ref, sdt_ref, dstc_ref):
    src3 = ei_ref[0, :].reshape(E // CH, CH)
    dst3 = ei_ref[1, :].reshape(E // CH, CH)
    sdt_ref[:E // CH, :] = src3 | lax.shift_left(dst3, 16)
    dstc_ref[:E // CH, :] = dst3
    nsl = EP // CH - E // CH
    row = lax.broadcasted_iota(jnp.int32, (nsl, CH), 0)
    col = lax.broadcasted_iota(jnp.int32, (nsl, CH), 1)
    slop = N + lax.rem(row * CH + col, NSLOP)
    sdt_ref[E // CH:, :] = slop | lax.shift_left(slop, 16)
    dstc_ref[E // CH:, :] = slop


def _tc_pre_body(degp_ref, t_ref, z_ref, wt1_ref, bt1_ref, wt2_ref, bt2_ref,
                 wenc_ref, xws1_ref, dinv_ref, temb_ref):
    deg = degp_ref[0, :] + degp_ref[1, :] + 1.0
    dinv_all = lax.rsqrt(deg)
    dinv = dinv_all[:N].reshape(N, 1)
    dinv_ref[...] = dinv
    t_in = t_ref[...].astype(jnp.float32)
    h = t_in * wt1_ref[...] + bt1_ref[...]
    h = 0.5 * h * (1.0 + lax.erf(h * 0.7071067811865476))
    temb_ref[...] = jnp.dot(h, wt2_ref[...],
                            preferred_element_type=jnp.float32) + bt2_ref[...]
    xw1 = jnp.dot(z_ref[...], wenc_ref[...],
                  preferred_element_type=jnp.float32)
    xws1_ref[:N, :] = xw1 * dinv
    xws1_ref[N:, :] = jnp.zeros((N_PAD - N, D), jnp.float32)


def _tc_mid_body(p_ref, xws1_ref, dinv_ref, temb_ref, benc_ref, wdec_ref,
                 xws2_ref):
    dinv = dinv_ref[...]
    agg = p_ref[0, :N, :] + p_ref[1, :N, :] + xws1_ref[:N, :]
    pre = dinv * agg + benc_ref[...] + temb_ref[...]
    h1 = jnp.where(pre > 0, pre, jnp.exp(jnp.minimum(pre, 0.0)) - 1.0)
    xw2 = jnp.dot(h1, wdec_ref[...], preferred_element_type=jnp.float32)
    xws2_ref[:N, :] = xw2 * dinv
    xws2_ref[N:, :] = jnp.zeros((N_PAD - N, D), jnp.float32)


def _tc_post_body(q_ref, xws2_ref, dinv_ref, bdec_ref, out_ref):
    agg = q_ref[0, :N, :] + q_ref[1, :N, :] + xws2_ref[:N, :]
    out_ref[...] = dinv_ref[...] * agg + bdec_ref[...]


def kernel(z, edge_index, t, W_t1, b_t1, W_t2, b_t2, W_enc, b_enc, W_dec,
           b_dec):
    ei = (edge_index if edge_index.dtype == jnp.int32
          else edge_index.astype(jnp.int32))
    sdt, dstc = pl.pallas_call(
        _tc_pack_body,
        out_shape=(
            jax.ShapeDtypeStruct((EP // CH, CH), jnp.int32),
            jax.ShapeDtypeStruct((EP // CH, CH), jnp.int32),
        ),
    )(ei)

    ones_ch = jnp.ones((CH,), jnp.float32)
    zvec = jnp.zeros((ROWS_PER_TILE,), jnp.float32)
    zrows = jnp.zeros((CH, D), jnp.float32)

    degp = _sc_degree(dstc, ones_ch, zvec).reshape(NC, N_PAD)

    xws1, dinv, temb = pl.pallas_call(
        _tc_pre_body,
        out_shape=(
            jax.ShapeDtypeStruct((N_PAD, D), jnp.float32),
            jax.ShapeDtypeStruct((N, 1), jnp.float32),
            jax.ShapeDtypeStruct((N, D), jnp.float32),
        ),
    )(degp, t.astype(jnp.int32).reshape(N, 1), z, W_t1, b_t1.reshape(1, D),
      W_t2, b_t2.reshape(1, D), W_enc)

    p = _sc_aggregate(xws1, sdt, zrows)

    xws2 = pl.pallas_call(
        _tc_mid_body,
        out_shape=jax.ShapeDtypeStruct((N_PAD, D), jnp.float32),
    )(p, xws1, dinv, temb, b_enc.reshape(1, D), W_dec)

    q = _sc_aggregate(xws2, sdt, zrows)

    out = pl.pallas_call(
        _tc_post_body,
        out_shape=jax.ShapeDtypeStruct((N, D), jnp.float32),
    )(q, xws2, dinv, b_dec.reshape(1, D))
    return out
